# baseline (speedup 1.0000x reference)
import functools

import jax
import jax.numpy as jnp
from jax.experimental import pallas as pl

NA = 3
NO = 85


def _head_kernel(x_ref, w_ref, b_ref, o_ref):
    res = jax.lax.dot_general(
        x_ref[0], w_ref[0],
        dimension_numbers=(((0,), (0,)), ((), ())),
        preferred_element_type=jnp.float32,
        precision=jax.lax.Precision.HIGHEST,
    )
    o_ref[0, 0] = res + b_ref[0]


@functools.partial(jax.jit, static_argnames=())
def _head(x, W, b):
    bs, C, ny, nx = x.shape
    hw = ny * nx
    xr = x.reshape(bs, C, hw)
    wt = W.reshape(NA, NO, C).transpose(0, 2, 1)
    br = b.reshape(NA, 1, NO)
    out = pl.pallas_call(
        _head_kernel,
        grid=(bs, NA),
        in_specs=[
            pl.BlockSpec((1, C, hw), lambda bidx, a: (bidx, 0, 0)),
            pl.BlockSpec((1, C, NO), lambda bidx, a: (a, 0, 0)),
            pl.BlockSpec((1, 1, NO), lambda bidx, a: (a, 0, 0)),
        ],
        out_specs=pl.BlockSpec((1, 1, hw, NO), lambda bidx, a: (bidx, a, 0, 0)),
        out_shape=jax.ShapeDtypeStruct((bs, NA, hw, NO), jnp.float32),
    )(xr, wt, br)
    return out.reshape(bs, NA, ny, nx, NO)


def kernel(x0, x1, x2, W0, b0, W1, b1, W2, b2):
    return (_head(x0, W0, b0), _head(x1, W1, b1), _head(x2, W2, b2))


# speedup vs baseline: 1.4939x; 1.4939x over previous
import functools

import jax
import jax.numpy as jnp
from jax.experimental import pallas as pl

NA = 3
NO = 85


def _head_kernel(x_ref, w_ref, b_ref, o_ref):
    res = jax.lax.dot_general(
        x_ref[0], w_ref[0],
        dimension_numbers=(((0,), (0,)), ((), ())),
        preferred_element_type=jnp.float32,
    )
    o_ref[0, 0] = res + b_ref[0]


@functools.partial(jax.jit, static_argnames=())
def _head(x, W, b):
    bs, C, ny, nx = x.shape
    hw = ny * nx
    xr = x.reshape(bs, C, hw)
    wt = W.reshape(NA, NO, C).transpose(0, 2, 1)
    br = b.reshape(NA, 1, NO)
    out = pl.pallas_call(
        _head_kernel,
        grid=(bs, NA),
        in_specs=[
            pl.BlockSpec((1, C, hw), lambda bidx, a: (bidx, 0, 0)),
            pl.BlockSpec((1, C, NO), lambda bidx, a: (a, 0, 0)),
            pl.BlockSpec((1, 1, NO), lambda bidx, a: (a, 0, 0)),
        ],
        out_specs=pl.BlockSpec((1, 1, hw, NO), lambda bidx, a: (bidx, a, 0, 0)),
        out_shape=jax.ShapeDtypeStruct((bs, NA, hw, NO), jnp.float32),
    )(xr, wt, br)
    return out.reshape(bs, NA, ny, nx, NO)


def kernel(x0, x1, x2, W0, b0, W1, b1, W2, b2):
    return (_head(x0, W0, b0), _head(x1, W1, b1), _head(x2, W2, b2))
